# baseline (speedup 1.0000x reference)
import functools

import jax
import jax.numpy as jnp
from jax import lax
from jax.experimental import pallas as pl
from jax.experimental.pallas import tpu as pltpu
from jax.experimental.pallas import tpu_sc as plsc

B = 16384
K = 16
NC = 2
NS = 16
L = 16
NW = NC * NS
BPW = B // NW
IDX_CH = 128
N_IDX = BPW // IDX_CH
CHUNKS = BPW // L


def _mf_body(uid_hbm, iid_hbm, umat_hbm, imat_hbm, out_hbm,
             uidx_v, iidx_v, urows_v, vrows_v, out_v, sem_u, sem_i):
    wid = lax.axis_index("s") * NC + lax.axis_index("c")
    base = wid * BPW

    for j in range(N_IDX):
        pltpu.sync_copy(uid_hbm.at[pl.ds(base + j * IDX_CH, IDX_CH)],
                        uidx_v.at[j])
        pltpu.sync_copy(iid_hbm.at[pl.ds(base + j * IDX_CH, IDX_CH)],
                        iidx_v.at[j])
    copies = []
    for j in range(N_IDX):
        copies.append(pltpu.async_copy(
            umat_hbm.at[uidx_v.at[j]],
            urows_v.at[pl.ds(j * IDX_CH, IDX_CH)], sem_u))
        copies.append(pltpu.async_copy(
            imat_hbm.at[iidx_v.at[j]],
            vrows_v.at[pl.ds(j * IDX_CH, IDX_CH)], sem_i))
    for c in copies:
        c.wait()

    def chunk_body(c, carry):
        row0 = pl.multiple_of(c * L, L)
        rows = row0 + lax.iota(jnp.int32, L)
        acc = jnp.zeros((L,), jnp.float32)
        for k in range(K):
            col = jnp.full((L,), k, jnp.int32)
            uu = plsc.load_gather(urows_v, [rows, col])
            vv = plsc.load_gather(vrows_v, [rows, col])
            acc = acc + uu * vv
        out_v[pl.ds(row0, L)] = acc
        return carry

    lax.fori_loop(0, CHUNKS, chunk_body, 0)
    pltpu.sync_copy(out_v, out_hbm.at[pl.ds(base, BPW)])


@functools.partial(jax.jit, donate_argnums=())
def kernel(uid, iid, user_mat, item_mat):
    uid = uid.astype(jnp.int32)
    iid = iid.astype(jnp.int32)
    mesh = plsc.VectorSubcoreMesh(core_axis_name="c", subcore_axis_name="s")
    run = pl.kernel(
        _mf_body, mesh=mesh,
        compiler_params=pltpu.CompilerParams(
            needs_layout_passes=False, use_tc_tiling_on_sc=False),
        out_type=jax.ShapeDtypeStruct((B,), jnp.float32),
        scratch_types=[
            pltpu.VMEM((N_IDX, IDX_CH), jnp.int32),
            pltpu.VMEM((N_IDX, IDX_CH), jnp.int32),
            pltpu.VMEM((BPW, K), jnp.float32),
            pltpu.VMEM((BPW, K), jnp.float32),
            pltpu.VMEM((BPW,), jnp.float32),
            pltpu.SemaphoreType.DMA,
            pltpu.SemaphoreType.DMA,
        ],
    )
    return run(uid, iid, user_mat, item_mat)


# speedup vs baseline: 6.3110x; 6.3110x over previous
import functools

import jax
import jax.numpy as jnp
from jax import lax
from jax.experimental import pallas as pl
from jax.experimental.pallas import tpu as pltpu
from jax.experimental.pallas import tpu_sc as plsc

B = 16384
K = 16
L = 16
NC = 2
NS = 16
NW = NC * NS
BPW = B // NW
G = 8
NG = BPW // G
SLOT_W = G * 128


def _mf_body(uid_hbm, iid_hbm, ut_hbm, vt_hbm, out_hbm,
             uidx_v, iidx_v, us0, us1, vs0, vs1, um_v, vm_v, out_v,
             sem_u0, sem_u1, sem_v0, sem_v1):
    wid = lax.axis_index("s") * NC + lax.axis_index("c")
    base = wid * BPW

    pltpu.sync_copy(uid_hbm.at[pl.ds(base, BPW)], uidx_v.at[pl.ds(0, BPW)])
    pltpu.sync_copy(iid_hbm.at[pl.ds(base, BPW)], iidx_v.at[pl.ds(0, BPW)])

    uslots = (us0, us1)
    vslots = (vs0, vs1)
    usems = (sem_u0, sem_u1)
    vsems = (sem_v0, sem_v1)

    def issue_group(g, parity):
        uvec = uidx_v[pl.ds(pl.multiple_of(g * G, G), L)]
        vvec = iidx_v[pl.ds(pl.multiple_of(g * G, G), L)]
        us = uslots[parity]
        vs = vslots[parity]
        for i in range(G):
            uoff = pl.multiple_of((uvec[i] // 128) * 128, 128)
            voff = pl.multiple_of((vvec[i] // 128) * 128, 128)
            pltpu.async_copy(ut_hbm.at[:, pl.ds(uoff, 128)],
                             us.at[:, pl.ds(i * 128, 128)], usems[parity])
            pltpu.async_copy(vt_hbm.at[:, pl.ds(voff, 128)],
                             vs.at[:, pl.ds(i * 128, 128)], vsems[parity])

    def drain_group(parity):
        pltpu.make_async_copy(ut_hbm.at[:, pl.ds(0, SLOT_W)],
                              uslots[parity], usems[parity]).wait()
        pltpu.make_async_copy(vt_hbm.at[:, pl.ds(0, SLOT_W)],
                              vslots[parity], vsems[parity]).wait()

    def extract_group(g, parity):
        uvec = uidx_v[pl.ds(pl.multiple_of(g * G, G), L)]
        vvec = iidx_v[pl.ds(pl.multiple_of(g * G, G), L)]
        us = uslots[parity]
        vs = vslots[parity]
        rows = lax.iota(jnp.int32, L)
        half = (g % 2) * G
        for i in range(G):
            ul = uvec[i] % 128
            vl = vvec[i] % 128
            ucol = jnp.full((L,), i * 128, jnp.int32) + ul
            vcol = jnp.full((L,), i * 128, jnp.int32) + vl
            uvals = plsc.load_gather(us, [rows, ucol])
            vvals = plsc.load_gather(vs, [rows, vcol])
            cidx = rows * L + (half + i)
            plsc.store_scatter(um_v, [cidx], uvals)
            plsc.store_scatter(vm_v, [cidx], vvals)

    def dot_chunk(c):
        acc = jnp.zeros((L,), jnp.float32)
        for k in range(K):
            acc = acc + um_v[pl.ds(k * L, L)] * vm_v[pl.ds(k * L, L)]
        out_v[pl.ds(pl.multiple_of(c * L, L), L)] = acc

    issue_group(0, 0)

    def group_body(g, carry):
        parity = lax.rem(g, 2)

        @pl.when(g + 1 < NG)
        def _():
            @pl.when(parity == 0)
            def _():
                issue_group(g + 1, 1)

            @pl.when(parity == 1)
            def _():
                issue_group(g + 1, 0)

        @pl.when(parity == 0)
        def _():
            drain_group(0)
            extract_group(g, 0)

        @pl.when(parity == 1)
        def _():
            drain_group(1)
            extract_group(g, 1)
            dot_chunk(g // 2)

        return carry

    lax.fori_loop(0, NG, group_body, 0)
    pltpu.sync_copy(out_v, out_hbm.at[pl.ds(base, BPW)])


@functools.partial(jax.jit, donate_argnums=())
def kernel(uid, iid, user_mat, item_mat):
    uid = uid.astype(jnp.int32)
    iid = iid.astype(jnp.int32)
    mesh = plsc.VectorSubcoreMesh(core_axis_name="c", subcore_axis_name="s")
    run = pl.kernel(
        _mf_body, mesh=mesh,
        compiler_params=pltpu.CompilerParams(
            needs_layout_passes=False, disable_bounds_checks=True),
        out_type=jax.ShapeDtypeStruct((B,), jnp.float32),
        scratch_types=[
            pltpu.VMEM((BPW + L,), jnp.int32),
            pltpu.VMEM((BPW + L,), jnp.int32),
            pltpu.VMEM((K, SLOT_W), jnp.float32),
            pltpu.VMEM((K, SLOT_W), jnp.float32),
            pltpu.VMEM((K, SLOT_W), jnp.float32),
            pltpu.VMEM((K, SLOT_W), jnp.float32),
            pltpu.VMEM((K * L,), jnp.float32),
            pltpu.VMEM((K * L,), jnp.float32),
            pltpu.VMEM((BPW,), jnp.float32),
            pltpu.SemaphoreType.DMA,
            pltpu.SemaphoreType.DMA,
            pltpu.SemaphoreType.DMA,
            pltpu.SemaphoreType.DMA,
        ],
    )
    return run(uid, iid, user_mat.T, item_mat.T)


# speedup vs baseline: 6.8419x; 1.0841x over previous
import functools

import jax
import jax.numpy as jnp
from jax import lax
from jax.experimental import pallas as pl
from jax.experimental.pallas import tpu as pltpu
from jax.experimental.pallas import tpu_sc as plsc

B = 16384
K = 16
L = 16
NC = 2
NS = 16
NW = NC * NS
BPW = B // NW
G = 8
NG = BPW // G
SLOT_W = G * 128


def _mf_body(uid_hbm, iid_hbm, ut_hbm, vt_hbm, out_hbm,
             uidx_v, iidx_v, us0, us1, us2, vs0, vs1, vs2, um_v, vm_v, out_v,
             sem_u0, sem_u1, sem_u2, sem_v0, sem_v1, sem_v2):
    wid = lax.axis_index("s") * NC + lax.axis_index("c")
    base = wid * BPW

    pltpu.sync_copy(uid_hbm.at[pl.ds(base, BPW)], uidx_v.at[pl.ds(0, BPW)])
    pltpu.sync_copy(iid_hbm.at[pl.ds(base, BPW)], iidx_v.at[pl.ds(0, BPW)])

    uslots = (us0, us1, us2)
    vslots = (vs0, vs1, vs2)
    usems = (sem_u0, sem_u1, sem_u2)
    vsems = (sem_v0, sem_v1, sem_v2)

    def issue_group(g, parity):
        uvec = uidx_v[pl.ds(pl.multiple_of(g * G, G), L)]
        vvec = iidx_v[pl.ds(pl.multiple_of(g * G, G), L)]
        us = uslots[parity]
        vs = vslots[parity]
        for i in range(G):
            uoff = pl.multiple_of((uvec[i] // 128) * 128, 128)
            voff = pl.multiple_of((vvec[i] // 128) * 128, 128)
            pltpu.async_copy(ut_hbm.at[:, pl.ds(uoff, 128)],
                             us.at[:, pl.ds(i * 128, 128)], usems[parity])
            pltpu.async_copy(vt_hbm.at[:, pl.ds(voff, 128)],
                             vs.at[:, pl.ds(i * 128, 128)], vsems[parity])

    def drain_group(parity):
        pltpu.make_async_copy(ut_hbm.at[:, pl.ds(0, SLOT_W)],
                              uslots[parity], usems[parity]).wait()
        pltpu.make_async_copy(vt_hbm.at[:, pl.ds(0, SLOT_W)],
                              vslots[parity], vsems[parity]).wait()

    def extract_group(g, parity):
        uvec = uidx_v[pl.ds(pl.multiple_of(g * G, G), L)]
        vvec = iidx_v[pl.ds(pl.multiple_of(g * G, G), L)]
        us = uslots[parity]
        vs = vslots[parity]
        rows = lax.iota(jnp.int32, L)
        half = (g % 2) * G
        for i in range(G):
            ul = uvec[i] % 128
            vl = vvec[i] % 128
            ucol = jnp.full((L,), i * 128, jnp.int32) + ul
            vcol = jnp.full((L,), i * 128, jnp.int32) + vl
            uvals = plsc.load_gather(us, [rows, ucol])
            vvals = plsc.load_gather(vs, [rows, vcol])
            cidx = rows * L + (half + i)
            plsc.store_scatter(um_v, [cidx], uvals)
            plsc.store_scatter(vm_v, [cidx], vvals)

    def dot_chunk(c):
        acc = jnp.zeros((L,), jnp.float32)
        for k in range(K):
            acc = acc + um_v[pl.ds(k * L, L)] * vm_v[pl.ds(k * L, L)]
        out_v[pl.ds(pl.multiple_of(c * L, L), L)] = acc

    issue_group(0, 0)
    issue_group(1, 1)

    def group_body(g, carry):
        parity = lax.rem(g, 3)

        for p in range(3):
            @pl.when((parity == p) & (g + 2 < NG))
            def _():
                issue_group(g + 2, (p + 2) % 3)

        for p in range(3):
            @pl.when(parity == p)
            def _():
                drain_group(p)
                extract_group(g, p)

        @pl.when(lax.rem(g, 2) == 1)
        def _():
            dot_chunk(g // 2)

        return carry

    lax.fori_loop(0, NG, group_body, 0)
    pltpu.sync_copy(out_v, out_hbm.at[pl.ds(base, BPW)])


@functools.partial(jax.jit, donate_argnums=())
def kernel(uid, iid, user_mat, item_mat):
    uid = uid.astype(jnp.int32)
    iid = iid.astype(jnp.int32)
    mesh = plsc.VectorSubcoreMesh(core_axis_name="c", subcore_axis_name="s")
    run = pl.kernel(
        _mf_body, mesh=mesh,
        compiler_params=pltpu.CompilerParams(
            needs_layout_passes=False, disable_bounds_checks=True),
        out_type=jax.ShapeDtypeStruct((B,), jnp.float32),
        scratch_types=[
            pltpu.VMEM((BPW + L,), jnp.int32),
            pltpu.VMEM((BPW + L,), jnp.int32),
            pltpu.VMEM((K, SLOT_W), jnp.float32),
            pltpu.VMEM((K, SLOT_W), jnp.float32),
            pltpu.VMEM((K, SLOT_W), jnp.float32),
            pltpu.VMEM((K, SLOT_W), jnp.float32),
            pltpu.VMEM((K, SLOT_W), jnp.float32),
            pltpu.VMEM((K, SLOT_W), jnp.float32),
            pltpu.VMEM((K * L,), jnp.float32),
            pltpu.VMEM((K * L,), jnp.float32),
            pltpu.VMEM((BPW,), jnp.float32),
            pltpu.SemaphoreType.DMA,
            pltpu.SemaphoreType.DMA,
            pltpu.SemaphoreType.DMA,
            pltpu.SemaphoreType.DMA,
            pltpu.SemaphoreType.DMA,
            pltpu.SemaphoreType.DMA,
        ],
    )
    return run(uid, iid, user_mat.T, item_mat.T)


# speedup vs baseline: 6.8760x; 1.0050x over previous
import functools

import jax
import jax.numpy as jnp
from jax import lax
from jax.experimental import pallas as pl
from jax.experimental.pallas import tpu as pltpu
from jax.experimental.pallas import tpu_sc as plsc

B = 16384
K = 16
L = 16
NC = 2
NS = 16
NW = NC * NS
BPW = B // NW
G = 8
NG = BPW // G
SLOT_W = G * 128


def _mf_body(uid_hbm, iid_hbm, ut_hbm, vt_hbm, out_hbm,
             uidx_v, iidx_v, us0, us1, us2, vs0, vs1, vs2, um_v, vm_v, out_v,
             sem_u0, sem_u1, sem_u2, sem_v0, sem_v1, sem_v2):
    wid = lax.axis_index("s") * NC + lax.axis_index("c")
    base = wid * BPW

    pltpu.sync_copy(uid_hbm.at[pl.ds(base, BPW)], uidx_v.at[pl.ds(0, BPW)])
    pltpu.sync_copy(iid_hbm.at[pl.ds(base, BPW)], iidx_v.at[pl.ds(0, BPW)])

    uslots = (us0, us1, us2)
    vslots = (vs0, vs1, vs2)
    usems = (sem_u0, sem_u1, sem_u2)
    vsems = (sem_v0, sem_v1, sem_v2)

    def issue_group(g, parity):
        uvec = uidx_v[pl.ds(pl.multiple_of(g * G, G), L)]
        vvec = iidx_v[pl.ds(pl.multiple_of(g * G, G), L)]
        us = uslots[parity]
        vs = vslots[parity]
        for i in range(G):
            uoff = pl.multiple_of((uvec[i] // 128) * 128, 128)
            voff = pl.multiple_of((vvec[i] // 128) * 128, 128)
            for h in range(2):
                pltpu.async_copy(
                    ut_hbm.at[pl.ds(h * 8, 8), pl.ds(uoff, 128)],
                    us.at[pl.ds(h * 8, 8), pl.ds(i * 128, 128)],
                    usems[parity])
                pltpu.async_copy(
                    vt_hbm.at[pl.ds(h * 8, 8), pl.ds(voff, 128)],
                    vs.at[pl.ds(h * 8, 8), pl.ds(i * 128, 128)],
                    vsems[parity])

    def drain_group(parity):
        pltpu.make_async_copy(ut_hbm.at[:, pl.ds(0, SLOT_W)],
                              uslots[parity], usems[parity]).wait()
        pltpu.make_async_copy(vt_hbm.at[:, pl.ds(0, SLOT_W)],
                              vslots[parity], vsems[parity]).wait()

    def extract_group(g, parity):
        uvec = uidx_v[pl.ds(pl.multiple_of(g * G, G), L)]
        vvec = iidx_v[pl.ds(pl.multiple_of(g * G, G), L)]
        us = uslots[parity]
        vs = vslots[parity]
        rows = lax.iota(jnp.int32, L)
        half = (g % 2) * G
        for i in range(G):
            ul = uvec[i] % 128
            vl = vvec[i] % 128
            ucol = jnp.full((L,), i * 128, jnp.int32) + ul
            vcol = jnp.full((L,), i * 128, jnp.int32) + vl
            uvals = plsc.load_gather(us, [rows, ucol])
            vvals = plsc.load_gather(vs, [rows, vcol])
            cidx = rows * L + (half + i)
            plsc.store_scatter(um_v, [cidx], uvals)
            plsc.store_scatter(vm_v, [cidx], vvals)

    def dot_chunk(c):
        acc = jnp.zeros((L,), jnp.float32)
        for k in range(K):
            acc = acc + um_v[pl.ds(k * L, L)] * vm_v[pl.ds(k * L, L)]
        out_v[pl.ds(pl.multiple_of(c * L, L), L)] = acc

    issue_group(0, 0)
    issue_group(1, 1)

    def group_body(g, carry):
        parity = lax.rem(g, 3)

        for p in range(3):
            @pl.when((parity == p) & (g + 2 < NG))
            def _():
                issue_group(g + 2, (p + 2) % 3)

        for p in range(3):
            @pl.when(parity == p)
            def _():
                drain_group(p)
                extract_group(g, p)

        @pl.when(lax.rem(g, 2) == 1)
        def _():
            dot_chunk(g // 2)

        return carry

    lax.fori_loop(0, NG, group_body, 0)
    pltpu.sync_copy(out_v, out_hbm.at[pl.ds(base, BPW)])


@functools.partial(jax.jit, donate_argnums=())
def kernel(uid, iid, user_mat, item_mat):
    uid = uid.astype(jnp.int32)
    iid = iid.astype(jnp.int32)
    mesh = plsc.VectorSubcoreMesh(core_axis_name="c", subcore_axis_name="s")
    run = pl.kernel(
        _mf_body, mesh=mesh,
        compiler_params=pltpu.CompilerParams(
            needs_layout_passes=False, disable_bounds_checks=True),
        out_type=jax.ShapeDtypeStruct((B,), jnp.float32),
        scratch_types=[
            pltpu.VMEM((BPW + L,), jnp.int32),
            pltpu.VMEM((BPW + L,), jnp.int32),
            pltpu.VMEM((K, SLOT_W), jnp.float32),
            pltpu.VMEM((K, SLOT_W), jnp.float32),
            pltpu.VMEM((K, SLOT_W), jnp.float32),
            pltpu.VMEM((K, SLOT_W), jnp.float32),
            pltpu.VMEM((K, SLOT_W), jnp.float32),
            pltpu.VMEM((K, SLOT_W), jnp.float32),
            pltpu.VMEM((K * L,), jnp.float32),
            pltpu.VMEM((K * L,), jnp.float32),
            pltpu.VMEM((BPW,), jnp.float32),
            pltpu.SemaphoreType.DMA,
            pltpu.SemaphoreType.DMA,
            pltpu.SemaphoreType.DMA,
            pltpu.SemaphoreType.DMA,
            pltpu.SemaphoreType.DMA,
            pltpu.SemaphoreType.DMA,
        ],
    )
    return run(uid, iid, user_mat.T, item_mat.T)
